# baseline (speedup 1.0000x reference)
import functools

import jax
import jax.numpy as jnp
from jax import lax
from jax.experimental import pallas as pl
from jax.experimental.pallas import tpu as pltpu
from jax.experimental.pallas import tpu_sc as plsc

INF_ = 1e9
B_, H_, W_ = 32, 32, 32
S_ = 36
FLD_ = S_ * S_
A0_, A1_ = 0, FLD_
PK_ = 2 * FLD_
MAX_PAIRS_ = 128


def _sc_body(w_hbm, meta_hbm, paths_hbm, exp_hbm, wv, mv, pk, pv, ev, dv):
    wid = lax.axis_index("s") * 2 + lax.axis_index("c")
    pltpu.sync_copy(w_hbm.at[wid], wv)
    pltpu.sync_copy(meta_hbm.at[wid], mv)

    def load(base):
        return pk[pl.ds(base, 16)]

    lanes = lax.iota(jnp.int32, 16)
    gdims = lax.GatherDimensionNumbers(
        offset_dims=(), collapsed_slice_dims=(0,), start_index_map=(0,)
    )

    def lane_reduce(x, op):
        for s in (1, 2, 4, 8):
            perm = lax.gather(
                x, (lanes ^ s)[:, None], gdims, (1,),
                mode=lax.GatherScatterMode.PROMISE_IN_BOUNDS,
            )
            x = op(x, perm)
        return x

    def splat_i(s):
        return jnp.broadcast_to(s, (16,)).astype(jnp.int32)

    def pick(vec, r):
        return lax.gather(
            vec, splat_i(r)[:, None], gdims, (1,),
            mode=lax.GatherScatterMode.PROMISE_IN_BOUNDS,
        )

    def elem(ref, base):
        lo = base & ~15
        return pick(ref[pl.ds(lo, 16)], base & 15)

    def elem2(ref, row, col):
        lo = col & ~15
        return pick(ref[row, pl.ds(lo, 16)], col & 15)

    inf_v = jnp.full((16,), INF_, jnp.float32)

    def fill(i, carry):
        pk[pl.ds(i * 16, 16)] = inf_v
        return carry

    lax.fori_loop(0, PK_ // 16, fill, jnp.int32(0), unroll=18)

    meta = mv[pl.ds(0, 16)]
    sy, sx, ty, tx = meta[2], meta[3], meta[4], meta[5]
    pp_s, pp_t = meta[6], meta[7]

    w_s = elem2(wv, sy, sx)
    w_t = elem2(wv, ty, tx)
    for off, pp, val in ((A0_, pp_s, w_s), (A1_, pp_t, w_t)):
        lo = pp & ~15
        pk[pl.ds(off + lo, 16)] = jnp.where(lanes == (pp & 15), val, inf_v)

    def row3(so, y, h):
        b = so + y * S_ + 1 + h
        return (load(b - 1), load(b), load(b + 1))

    shr_idx = (lanes - 1) & 15
    shl_idx = (lanes + 1) & 15

    def perm(v, idx):
        return lax.gather(
            v, idx[:, None], gdims, (1,),
            mode=lax.GatherScatterMode.PROMISE_IN_BOUNDS,
        )

    def shr(v, fill):
        return jnp.where(lanes == 0, fill, perm(v, shr_idx))

    def shl(v, fill):
        return jnp.where(lanes == 15, fill, perm(v, shl_idx))

    def gs_sweep(down):
        y0 = 1 if down else H_
        step = 1 if down else -1
        init = tuple(
            row3(so, y0, 0) + row3(so, y0, 16) + (inf_v, inf_v)
            for so in (A0_, A1_)
        )

        def row(i, carry):
            acc, st = carry
            y = (1 + i) if down else (H_ - i)
            far = y + step
            nst = []
            wrow = (wv[y - 1, pl.ds(0, 16)], wv[y - 1, pl.ds(16, 16)])
            for so, win in zip((A0_, A1_), st):
                c_m0, c_00, c_p0, c_m1, c_01, c_p1, a_00, a_01 = win
                a_m0 = shr(a_00, inf_v)
                a_p0 = shl(a_00, pick(a_01, 0))
                a_m1 = shr(a_01, pick(a_00, 15))
                a_p1 = shl(a_01, inf_v)
                f_m0, f_00, f_p0 = row3(so, far, 0)
                f_m1, f_01, f_p1 = row3(so, far, 16)
                m0 = jnp.minimum(
                    jnp.minimum(jnp.minimum(a_m0, a_00), jnp.minimum(a_p0, c_m0)),
                    jnp.minimum(jnp.minimum(c_p0, f_m0), jnp.minimum(f_00, f_p0)),
                )
                m1 = jnp.minimum(
                    jnp.minimum(jnp.minimum(a_m1, a_01), jnp.minimum(a_p1, c_m1)),
                    jnp.minimum(jnp.minimum(c_p1, f_m1), jnp.minimum(f_01, f_p1)),
                )
                b0 = jnp.minimum(c_00, m0 + wrow[0])
                b1 = jnp.minimum(c_01, m1 + wrow[1])
                pk[pl.ds(so + y * S_ + 1, 16)] = b0
                pk[pl.ds(so + y * S_ + 17, 16)] = b1
                acc = jnp.maximum(acc, jnp.maximum(c_00 - b0, c_01 - b1))
                nst.append((f_m0, f_00, f_p0, f_m1, f_01, f_p1, b0, b1))
            return acc, tuple(nst)

        acc, _ = lax.fori_loop(
            0, H_, row, (jnp.zeros((16,), jnp.float32), init), unroll=8
        )
        return lane_reduce(acc, jnp.maximum)

    dv[pl.ds(0, 16)] = jnp.ones((16,), jnp.float32)

    def pair_body(i, carry):
        flag = dv[pl.ds(0, 16)]

        @pl.when(flag[0] > 0.0)
        def _():
            gs_sweep(True)
            dv[pl.ds(0, 16)] = gs_sweep(False)
        return carry

    lax.fori_loop(0, MAX_PAIRS_, pair_body, jnp.int32(0))

    total = elem(pk, A0_ + pp_t)
    thresh = total + 1e-4
    reach = jnp.float32(INF_ * 0.5)
    one = jnp.ones((16,), jnp.float32)
    zero = jnp.zeros((16,), jnp.float32)

    def out_row(y, carry):
        r = y * S_ + 1
        for h in (0, 16):
            ds = load(A0_ + r + h)
            dt = load(A1_ + r + h)
            wr = wv[y - 1, pl.ds(h, 16)]
            on = (ds + dt - wr <= thresh) & (ds < reach) & (dt < reach)
            pv[y - 1, pl.ds(h, 16)] = jnp.where(on, one, zero)
            ev[y - 1, pl.ds(h, 16)] = jnp.where(ds < reach, one, zero)
        return carry

    lax.fori_loop(1, 1 + H_, out_row, jnp.int32(0), unroll=8)

    pltpu.sync_copy(pv, paths_hbm.at[wid])
    pltpu.sync_copy(ev, exp_hbm.at[wid])


@jax.jit
def kernel(weights, source, target):
    w = weights.astype(jnp.float32)
    src = source.astype(jnp.int32)
    tgt = target.astype(jnp.int32)
    sy, sx = src // W_, src % W_
    ty, tx = tgt // W_, tgt % W_
    pp_s = (sy + 1) * S_ + sx + 1
    pp_t = (ty + 1) * S_ + tx + 1
    meta = jnp.pad(
        jnp.stack([src, tgt, sy, sx, ty, tx, pp_s, pp_t], axis=1),
        ((0, 0), (0, 8)),
    )

    mesh = plsc.VectorSubcoreMesh(
        core_axis_name="c", subcore_axis_name="s", num_cores=2, num_subcores=16
    )
    run = pl.kernel(
        _sc_body,
        out_type=(
            jax.ShapeDtypeStruct((B_, H_, W_), jnp.float32),
            jax.ShapeDtypeStruct((B_, H_, W_), jnp.float32),
        ),
        mesh=mesh,
        scratch_types=(
            pltpu.VMEM((H_, W_), jnp.float32),
            pltpu.VMEM((16,), jnp.int32),
            pltpu.VMEM((PK_,), jnp.float32),
            pltpu.VMEM((H_, W_), jnp.float32),
            pltpu.VMEM((H_, W_), jnp.float32),
            pltpu.VMEM((16,), jnp.float32),
        ),
    )
    return run(w, meta)


# speedup vs baseline: 1.0467x; 1.0467x over previous
import functools

import jax
import jax.numpy as jnp
from jax import lax
from jax.experimental import pallas as pl
from jax.experimental.pallas import tpu as pltpu
from jax.experimental.pallas import tpu_sc as plsc

INF_ = 1e9
B_, H_, W_ = 32, 32, 32
S_ = 36
FLD_ = S_ * S_
A0_, A1_ = 0, FLD_
PK_ = 2 * FLD_
MAX_PAIRS_ = 128


def _sc_body(w_hbm, meta_hbm, paths_hbm, exp_hbm, wv, mv, pk, pv, ev, dv,
             sem_w, sem_m, sem_p, sem_e):
    wid = lax.axis_index("s") * 2 + lax.axis_index("c")
    cp_w = pltpu.async_copy(w_hbm.at[wid], wv, sem_w)
    cp_m = pltpu.async_copy(meta_hbm.at[wid], mv, sem_m)

    def load(base):
        return pk[pl.ds(base, 16)]

    lanes = lax.iota(jnp.int32, 16)
    gdims = lax.GatherDimensionNumbers(
        offset_dims=(), collapsed_slice_dims=(0,), start_index_map=(0,)
    )

    def lane_reduce(x, op):
        for s in (1, 2, 4, 8):
            perm = lax.gather(
                x, (lanes ^ s)[:, None], gdims, (1,),
                mode=lax.GatherScatterMode.PROMISE_IN_BOUNDS,
            )
            x = op(x, perm)
        return x

    def splat_i(s):
        return jnp.broadcast_to(s, (16,)).astype(jnp.int32)

    def pick(vec, r):
        return lax.gather(
            vec, splat_i(r)[:, None], gdims, (1,),
            mode=lax.GatherScatterMode.PROMISE_IN_BOUNDS,
        )

    def elem(ref, base):
        lo = base & ~15
        return pick(ref[pl.ds(lo, 16)], base & 15)

    def elem2(ref, row, col):
        lo = col & ~15
        return pick(ref[row, pl.ds(lo, 16)], col & 15)

    inf_v = jnp.full((16,), INF_, jnp.float32)

    def fill(i, carry):
        pk[pl.ds(i * 16, 16)] = inf_v
        return carry

    lax.fori_loop(0, PK_ // 16, fill, jnp.int32(0), unroll=18)
    cp_w.wait()
    cp_m.wait()

    meta = mv[pl.ds(0, 16)]
    sy, sx, ty, tx = meta[2], meta[3], meta[4], meta[5]
    pp_s, pp_t = meta[6], meta[7]

    w_s = elem2(wv, sy, sx)
    w_t = elem2(wv, ty, tx)
    for off, pp, val in ((A0_, pp_s, w_s), (A1_, pp_t, w_t)):
        lo = pp & ~15
        pk[pl.ds(off + lo, 16)] = jnp.where(lanes == (pp & 15), val, inf_v)

    def row3(so, y, h):
        b = so + y * S_ + 1 + h
        return (load(b - 1), load(b), load(b + 1))

    shr_idx = (lanes - 1) & 15
    shl_idx = (lanes + 1) & 15

    def perm(v, idx):
        return lax.gather(
            v, idx[:, None], gdims, (1,),
            mode=lax.GatherScatterMode.PROMISE_IN_BOUNDS,
        )

    def shr(v, fill):
        return jnp.where(lanes == 0, fill, perm(v, shr_idx))

    def shl(v, fill):
        return jnp.where(lanes == 15, fill, perm(v, shl_idx))

    def gs_sweep(down):
        y0 = 1 if down else H_
        step = 1 if down else -1
        init = tuple(
            row3(so, y0, 0) + row3(so, y0, 16) + (inf_v, inf_v)
            for so in (A0_, A1_)
        )

        def row(i, carry):
            acc, st = carry
            y = (1 + i) if down else (H_ - i)
            far = y + step
            nst = []
            wrow = (wv[y - 1, pl.ds(0, 16)], wv[y - 1, pl.ds(16, 16)])
            for so, win in zip((A0_, A1_), st):
                c_m0, c_00, c_p0, c_m1, c_01, c_p1, a_00, a_01 = win
                a_m0 = shr(a_00, inf_v)
                a_p0 = shl(a_00, pick(a_01, 0))
                a_m1 = shr(a_01, pick(a_00, 15))
                a_p1 = shl(a_01, inf_v)
                f_m0, f_00, f_p0 = row3(so, far, 0)
                f_m1, f_01, f_p1 = row3(so, far, 16)
                m0 = jnp.minimum(
                    jnp.minimum(jnp.minimum(a_m0, a_00), jnp.minimum(a_p0, c_m0)),
                    jnp.minimum(jnp.minimum(c_p0, f_m0), jnp.minimum(f_00, f_p0)),
                )
                m1 = jnp.minimum(
                    jnp.minimum(jnp.minimum(a_m1, a_01), jnp.minimum(a_p1, c_m1)),
                    jnp.minimum(jnp.minimum(c_p1, f_m1), jnp.minimum(f_01, f_p1)),
                )
                b0 = jnp.minimum(c_00, m0 + wrow[0])
                b1 = jnp.minimum(c_01, m1 + wrow[1])
                pk[pl.ds(so + y * S_ + 1, 16)] = b0
                pk[pl.ds(so + y * S_ + 17, 16)] = b1
                acc = jnp.maximum(acc, jnp.maximum(c_00 - b0, c_01 - b1))
                nst.append((f_m0, f_00, f_p0, f_m1, f_01, f_p1, b0, b1))
            return acc, tuple(nst)

        acc, _ = lax.fori_loop(
            0, H_, row, (jnp.zeros((16,), jnp.float32), init), unroll=4
        )
        return lane_reduce(acc, jnp.maximum)

    dv[pl.ds(0, 16)] = jnp.ones((16,), jnp.float32)

    def pair_body(i, carry):
        flag = dv[pl.ds(0, 16)]

        @pl.when(flag[0] > 0.0)
        def _():
            gs_sweep(True)
            dv[pl.ds(0, 16)] = gs_sweep(False)
        return carry

    def chunk_body(i, carry):
        flag = dv[pl.ds(0, 16)]

        @pl.when(flag[0] > 0.0)
        def _():
            lax.fori_loop(0, 8, pair_body, jnp.int32(0))
        return carry

    lax.fori_loop(0, MAX_PAIRS_ // 8, chunk_body, jnp.int32(0))

    total = elem(pk, A0_ + pp_t)
    thresh = total + 1e-4
    reach = jnp.float32(INF_ * 0.5)
    one = jnp.ones((16,), jnp.float32)
    zero = jnp.zeros((16,), jnp.float32)

    def out_row(y, carry):
        r = y * S_ + 1
        for h in (0, 16):
            ds = load(A0_ + r + h)
            dt = load(A1_ + r + h)
            wr = wv[y - 1, pl.ds(h, 16)]
            on = (ds + dt - wr <= thresh) & (ds < reach) & (dt < reach)
            pv[y - 1, pl.ds(h, 16)] = jnp.where(on, one, zero)
            ev[y - 1, pl.ds(h, 16)] = jnp.where(ds < reach, one, zero)
        return carry

    lax.fori_loop(1, 1 + H_, out_row, jnp.int32(0), unroll=8)

    cp_p = pltpu.async_copy(pv, paths_hbm.at[wid], sem_p)
    cp_e = pltpu.async_copy(ev, exp_hbm.at[wid], sem_e)
    cp_p.wait()
    cp_e.wait()


@jax.jit
def kernel(weights, source, target):
    w = weights.astype(jnp.float32)
    src = source.astype(jnp.int32)
    tgt = target.astype(jnp.int32)
    sy, sx = src // W_, src % W_
    ty, tx = tgt // W_, tgt % W_
    pp_s = (sy + 1) * S_ + sx + 1
    pp_t = (ty + 1) * S_ + tx + 1
    meta = jnp.pad(
        jnp.stack([src, tgt, sy, sx, ty, tx, pp_s, pp_t], axis=1),
        ((0, 0), (0, 8)),
    )

    mesh = plsc.VectorSubcoreMesh(
        core_axis_name="c", subcore_axis_name="s", num_cores=2, num_subcores=16
    )
    run = pl.kernel(
        _sc_body,
        out_type=(
            jax.ShapeDtypeStruct((B_, H_, W_), jnp.float32),
            jax.ShapeDtypeStruct((B_, H_, W_), jnp.float32),
        ),
        mesh=mesh,
        scratch_types=(
            pltpu.VMEM((H_, W_), jnp.float32),
            pltpu.VMEM((16,), jnp.int32),
            pltpu.VMEM((PK_,), jnp.float32),
            pltpu.VMEM((H_, W_), jnp.float32),
            pltpu.VMEM((H_, W_), jnp.float32),
            pltpu.VMEM((16,), jnp.float32),
            pltpu.SemaphoreType.DMA,
            pltpu.SemaphoreType.DMA,
            pltpu.SemaphoreType.DMA,
            pltpu.SemaphoreType.DMA,
        ),
    )
    return run(w, meta)
